# Initial kernel scaffold; baseline (speedup 1.0000x reference)
import functools

import numpy as np
import jax
import jax.numpy as jnp
from jax import lax
from jax.experimental import pallas as pl
from jax.experimental.pallas import tpu as pltpu
from jax.experimental.pallas import tpu_sc as plsc

_VOCAB = 1_000_000
_D = 64
_H = 64
_J = 256
_B = 16384
_L = 50

_NC = 2
_NS = 16
_NW = _NC * _NS
_IPW = _B // _NW
_RPW = _IPW * _L
_CHUNK = 128
_NCH = _RPW // _CHUNK

_SEG_NP = (np.arange(_RPW, dtype=np.int32) // _L).reshape(_NCH, 1, _CHUNK)


def _sc_pool(x_flat, seg, table):
    mesh = plsc.VectorSubcoreMesh(core_axis_name="c", subcore_axis_name="s")

    @functools.partial(
        pl.kernel,
        out_type=jax.ShapeDtypeStruct((_B, _D), jnp.float32),
        mesh=mesh,
        scratch_types=[
            pltpu.VMEM((_RPW,), jnp.int32),
            pltpu.VMEM((_NCH, 1, _CHUNK), jnp.int32),
            pltpu.VMEM((_IPW, _D), jnp.float32),
            pltpu.VMEM((_CHUNK, _D), jnp.float32),
            pltpu.VMEM((_CHUNK, _D), jnp.float32),
            pltpu.SemaphoreType.DMA,
            pltpu.SemaphoreType.DMA,
        ],
    )
    def k(x_hbm, seg_hbm, table_hbm, out_hbm,
          idx_v, seg_v, acc_v, rows0, rows1, sem0, sem1):
        wid = lax.axis_index("s") * _NC + lax.axis_index("c")
        base_row = wid * _RPW
        pltpu.sync_copy(x_hbm.at[pl.ds(base_row, _RPW)], idx_v)
        pltpu.sync_copy(seg_hbm, seg_v)

        zeros16 = jnp.zeros((16,), jnp.float32)

        @pl.loop(0, _IPW)
        def _(i):
            for c in range(_D // 16):
                acc_v[i, pl.ds(c * 16, 16)] = zeros16

        rows = (rows0, rows1)
        sems = (sem0, sem1)

        def start(cc, b):
            pltpu.async_copy(
                table_hbm.at[idx_v.at[pl.ds(cc * _CHUNK, _CHUNK)]],
                rows[b], sems[b])

        def finish(cc, b):
            pltpu.make_async_copy(
                table_hbm.at[idx_v.at[pl.ds(cc * _CHUNK, _CHUNK)]],
                rows[b], sems[b]).wait()
            pltpu.sync_copy(rows[b], acc_v.at[seg_v.at[cc, 0]], add=True)

        start(0, 0)
        start(1, 1)

        @pl.loop(0, _NCH - 2, step=2)
        def _(c):
            for b in range(2):
                cc = c + b
                finish(cc, b)
                start(cc + 2, b)

        finish(_NCH - 2, 0)
        finish(_NCH - 1, 1)

        pltpu.sync_copy(acc_v, out_hbm.at[pl.ds(wid * _IPW, _IPW)])

    return k(x_flat, seg, table)


def _tc_mlp(pooled, w1t, b1, w2t, b2):
    blk = 1024

    def body(p_ref, w1_ref, b1_ref, w2_ref, b2_ref, o_ref):
        p = p_ref[...] * (1.0 / _L)
        h = jnp.dot(p, w1_ref[...], preferred_element_type=jnp.float32,
                    precision=lax.Precision.HIGHEST)
        h = jnp.maximum(h + b1_ref[...], 0.0)
        o = jnp.dot(h, w2_ref[...], preferred_element_type=jnp.float32,
                    precision=lax.Precision.HIGHEST) + b2_ref[...]
        nrm = jnp.sqrt(jnp.sum(o * o, axis=1, keepdims=True))
        o_ref[...] = o / jnp.maximum(nrm, 1e-12)

    return pl.pallas_call(
        body,
        grid=(_B // blk,),
        in_specs=[
            pl.BlockSpec((blk, _D), lambda i: (i, 0)),
            pl.BlockSpec((_D, _H), lambda i: (0, 0)),
            pl.BlockSpec((1, _H), lambda i: (0, 0)),
            pl.BlockSpec((_H, _J), lambda i: (0, 0)),
            pl.BlockSpec((1, _J), lambda i: (0, 0)),
        ],
        out_specs=pl.BlockSpec((blk, _J), lambda i: (i, 0)),
        out_shape=jax.ShapeDtypeStruct((_B, _J), jnp.float32),
    )(pooled, w1t, b1.reshape(1, _H), w2t, b2.reshape(1, _J))


def kernel(x, table, W1, b1, W2, b2):
    x_flat = x.reshape(-1).astype(jnp.int32)
    seg = jnp.asarray(_SEG_NP)
    pooled = _sc_pool(x_flat, seg, table)
    return _tc_mlp(pooled, W1.T, b1, W2.T, b2)


# speedup vs baseline: 2.5254x; 2.5254x over previous
import functools

import numpy as np
import jax
import jax.numpy as jnp
from jax import lax
from jax.experimental import pallas as pl
from jax.experimental.pallas import tpu as pltpu
from jax.experimental.pallas import tpu_sc as plsc

_VOCAB = 1_000_000
_D = 64
_H = 64
_J = 256
_B = 16384
_L = 50

_NC = 2
_NS = 16
_NW = _NC * _NS
_IPW = _B // _NW
_RPW = _IPW * _L
_CHUNK = 128
_NCH = _RPW // _CHUNK

_SEG_NP = (np.arange(_RPW, dtype=np.int32) // _L).reshape(_NCH, 1, _CHUNK)


def _sc_pool(x_flat, seg, table):
    mesh = plsc.VectorSubcoreMesh(core_axis_name="c", subcore_axis_name="s",
                                  num_cores=_NC, num_subcores=_NS)

    @functools.partial(
        pl.kernel,
        out_type=jax.ShapeDtypeStruct((_B, _D), jnp.float32),
        mesh=mesh,
        compiler_params=pltpu.CompilerParams(use_tc_tiling_on_sc=False),
        scratch_types=[
            pltpu.VMEM((_RPW,), jnp.int32),
            pltpu.VMEM((_NCH, 1, _CHUNK), jnp.int32),
            pltpu.VMEM_SHARED((_NS, _IPW, _D), jnp.float32),
            pltpu.VMEM((_CHUNK, _D), jnp.float32),
            pltpu.VMEM((_CHUNK, _D), jnp.float32),
            pltpu.SemaphoreType.DMA,
            pltpu.SemaphoreType.DMA,
        ],
    )
    def k(x_hbm, seg_hbm, table_hbm, out_hbm,
          idx_v, seg_v, acc_sh, rows0, rows1, sem0, sem1):
        cid = lax.axis_index("c")
        sid = lax.axis_index("s")
        wid = cid * _NS + sid
        base_row = wid * _RPW
        pltpu.sync_copy(x_hbm.at[pl.ds(base_row, _RPW)], idx_v)
        pltpu.sync_copy(seg_hbm, seg_v)
        acc_v = acc_sh.at[sid]

        zeros16 = jnp.zeros((16,), jnp.float32)

        @pl.loop(0, _CHUNK)
        def _(i):
            for c in range(_D // 16):
                rows0[i, pl.ds(c * 16, 16)] = zeros16

        for z in range(_IPW // _CHUNK):
            pltpu.sync_copy(rows0, acc_v.at[pl.ds(z * _CHUNK, _CHUNK)])

        rows = (rows0, rows1)
        sems = (sem0, sem1)

        def start(cc, b):
            pltpu.async_copy(
                table_hbm.at[idx_v.at[pl.ds(cc * _CHUNK, _CHUNK)]],
                rows[b], sems[b])

        def finish(cc, b):
            pltpu.make_async_copy(
                table_hbm.at[idx_v.at[pl.ds(cc * _CHUNK, _CHUNK)]],
                rows[b], sems[b]).wait()
            pltpu.sync_copy(rows[b], acc_v.at[seg_v.at[cc, 0]], add=True)

        start(0, 0)
        start(1, 1)

        @pl.loop(0, _NCH - 2, step=2)
        def _(c):
            for b in range(2):
                cc = c + b
                finish(cc, b)
                start(cc + 2, b)

        finish(_NCH - 2, 0)
        finish(_NCH - 1, 1)

        pltpu.sync_copy(acc_v, out_hbm.at[pl.ds(wid * _IPW, _IPW)])

    return k(x_flat, seg, table)


def _tc_mlp(pooled, w1t, b1, w2t, b2):
    blk = 1024

    def body(p_ref, w1_ref, b1_ref, w2_ref, b2_ref, o_ref):
        p = p_ref[...] * (1.0 / _L)
        h = jnp.dot(p, w1_ref[...], preferred_element_type=jnp.float32,
                    precision=lax.Precision.HIGHEST)
        h = jnp.maximum(h + b1_ref[...], 0.0)
        o = jnp.dot(h, w2_ref[...], preferred_element_type=jnp.float32,
                    precision=lax.Precision.HIGHEST) + b2_ref[...]
        nrm = jnp.sqrt(jnp.sum(o * o, axis=1, keepdims=True))
        o_ref[...] = o / jnp.maximum(nrm, 1e-12)

    return pl.pallas_call(
        body,
        grid=(_B // blk,),
        in_specs=[
            pl.BlockSpec((blk, _D), lambda i: (i, 0)),
            pl.BlockSpec((_D, _H), lambda i: (0, 0)),
            pl.BlockSpec((1, _H), lambda i: (0, 0)),
            pl.BlockSpec((_H, _J), lambda i: (0, 0)),
            pl.BlockSpec((1, _J), lambda i: (0, 0)),
        ],
        out_specs=pl.BlockSpec((blk, _J), lambda i: (i, 0)),
        out_shape=jax.ShapeDtypeStruct((_B, _J), jnp.float32),
    )(pooled, w1t, b1.reshape(1, _H), w2t, b2.reshape(1, _J))


def kernel(x, table, W1, b1, W2, b2):
    x_flat = x.reshape(-1).astype(jnp.int32)
    seg = jnp.asarray(_SEG_NP)
    pooled = _sc_pool(x_flat, seg, table)
    return _tc_mlp(pooled, W1.T, b1, W2.T, b2)
